# baseline (speedup 1.0000x reference)
import functools

import jax
import jax.numpy as jnp
from jax import lax
from jax.experimental import pallas as pl
from jax.experimental.pallas import tpu as pltpu
from jax.experimental.pallas import tpu_sc as plsc


def _sc_scalar_lookup(idx_flat, table):
    B = idx_flat.shape[0]
    V, D = table.shape
    mesh = plsc.ScalarSubcoreMesh(axis_name="c", num_cores=1)

    @functools.partial(
        pl.kernel,
        out_type=jax.ShapeDtypeStruct((B, D), jnp.float32),
        mesh=mesh,
        compiler_params=pltpu.CompilerParams(
            needs_layout_passes=False,
            skip_device_barrier=True,
            disable_bounds_checks=True,
            disable_semaphore_checks=True,
        ),
        scratch_types=[
            pltpu.SMEM((B,), jnp.int32),
            pltpu.SemaphoreType.DMA,
        ],
    )
    def body(idx_hbm, table_hbm, out_hbm, idx_s, sem):
        pltpu.sync_copy(idx_hbm, idx_s)
        copies = [
            pltpu.async_copy(
                table_hbm.at[pl.ds(idx_s[b], 1)], out_hbm.at[pl.ds(b, 1)], sem
            )
            for b in range(B)
        ]
        for c in copies:
            c.wait()

    return body(idx_flat, table)


def kernel(inputs, table):
    out = _sc_scalar_lookup(inputs.reshape(-1).astype(jnp.int32), table)
    return out.reshape(inputs.shape + (table.shape[1],))


# speedup vs baseline: 1.0312x; 1.0312x over previous
import functools

import jax
import jax.numpy as jnp
from jax import lax
from jax.experimental import pallas as pl
from jax.experimental.pallas import tpu as pltpu
from jax.experimental.pallas import tpu_sc as plsc


def _sc_scalar_lookup(idx_flat, table):
    B = idx_flat.shape[0]
    V, D = table.shape
    mesh = plsc.ScalarSubcoreMesh(axis_name="c", num_cores=1)

    @functools.partial(
        pl.kernel,
        out_type=jax.ShapeDtypeStruct((B, D), jnp.float32),
        mesh=mesh,
        compiler_params=pltpu.CompilerParams(needs_layout_passes=False),
        scratch_types=[
            pltpu.SMEM((B,), jnp.int32),
            pltpu.SMEM((V, D), jnp.float32),
            pltpu.SemaphoreType.DMA,
        ],
    )
    def body(idx_hbm, table_hbm, out_hbm, idx_s, tab_s, sem):
        ins = [
            pltpu.async_copy(idx_hbm, idx_s, sem),
            pltpu.async_copy(table_hbm, tab_s, sem),
        ]
        for c in ins:
            c.wait()
        outs = [
            pltpu.async_copy(
                tab_s.at[pl.ds(idx_s[b], 1)], out_hbm.at[pl.ds(b, 1)], sem
            )
            for b in range(B)
        ]
        for c in outs:
            c.wait()

    return body(idx_flat, table)


def kernel(inputs, table):
    out = _sc_scalar_lookup(inputs.reshape(-1).astype(jnp.int32), table)
    return out.reshape(inputs.shape + (table.shape[1],))
